# Initial kernel scaffold; baseline (speedup 1.0000x reference)
import functools

import jax
import jax.numpy as jnp
from jax import lax
from jax.experimental import pallas as pl
from jax.experimental.pallas import tpu as pltpu
from jax.experimental.pallas import tpu_sc as plsc

S, V, D, B, L = 26, 100000, 32, 4096, 20

NC, NS = 2, 16
NW = NC * NS
BPW = B // NW
RPW = BPW * L
CHUNK = 128
NCHUNK = RPW // CHUNK


def _sc_pool_kernel(tab_ref, idx_ref, out_ref, idx_v, rows_v, out_v, sem):
    wid = lax.axis_index("s") * NC + lax.axis_index("c")
    base = wid * BPW

    def slot_body(s, _):
        pltpu.sync_copy(idx_ref.at[s, wid], idx_v)
        s_off = s * V

        def fire(c, _):
            for i in range(CHUNK // 16):
                sl = pl.ds(i * 16, 16)
                idx_v[c, sl] = idx_v[c, sl] + s_off
            pltpu.make_async_copy(
                tab_ref.at[idx_v.at[c]],
                rows_v.at[pl.ds(c * CHUNK, CHUNK)],
                sem,
            ).start()
            return 0

        lax.fori_loop(0, NCHUNK, fire, 0)

        def drain(c, _):
            pltpu.make_async_copy(
                tab_ref.at[idx_v.at[c]],
                rows_v.at[pl.ds(c * CHUNK, CHUNK)],
                sem,
            ).wait()
            return 0

        lax.fori_loop(0, NCHUNK, drain, 0)

        def pool(b, _):
            r0 = b * L
            lo = rows_v[r0, pl.ds(0, 16)]
            hi = rows_v[r0, pl.ds(16, 16)]
            for r in range(1, L):
                lo = lo + rows_v[r0 + r, pl.ds(0, 16)]
                hi = hi + rows_v[r0 + r, pl.ds(16, 16)]
            out_v[b, pl.ds(0, 16)] = lo
            out_v[b, pl.ds(16, 16)] = hi
            return 0

        lax.fori_loop(0, BPW, pool, 0)
        pltpu.sync_copy(out_v, out_ref.at[s, pl.ds(base, BPW)])
        return 0

    lax.fori_loop(0, S, slot_body, 0)


@jax.jit
def _sc_pool(tab_flat, idx4):
    mesh = plsc.VectorSubcoreMesh(core_axis_name="c", subcore_axis_name="s")
    return pl.kernel(
        _sc_pool_kernel,
        out_type=jax.ShapeDtypeStruct((S, B, D), jnp.float32),
        mesh=mesh,
        scratch_types=[
            pltpu.VMEM((NCHUNK, CHUNK), jnp.int32),
            pltpu.VMEM((RPW, D), jnp.float32),
            pltpu.VMEM((BPW, D), jnp.float32),
            pltpu.SemaphoreType.DMA,
        ],
    )(tab_flat, idx4)


def _mlp_kernel(x_ref, w1_ref, b1_ref, w2_ref, b2_ref, w3_ref, b3_ref,
                w4_ref, b4_ref, wo_ref, bo_ref, o_ref):
    x = x_ref[...]
    w1 = w1_ref[...]
    h = jnp.dot(x[0], w1[0], preferred_element_type=jnp.float32)
    for s in range(1, S):
        h = h + jnp.dot(x[s], w1[s], preferred_element_type=jnp.float32)
    h = jnp.maximum(h + b1_ref[...], 0.0)
    h = jnp.maximum(jnp.dot(h, w2_ref[...], preferred_element_type=jnp.float32)
                    + b2_ref[...], 0.0)
    h = jnp.maximum(jnp.dot(h, w3_ref[...], preferred_element_type=jnp.float32)
                    + b3_ref[...], 0.0)
    h = jnp.maximum(jnp.dot(h, w4_ref[...], preferred_element_type=jnp.float32)
                    + b4_ref[...], 0.0)
    y = jnp.dot(h, wo_ref[...], preferred_element_type=jnp.float32) + bo_ref[...]
    o_ref[...] = jax.nn.sigmoid(y)


def _mlp(pooled, w1r, b1, w2, b2, w3, b3, w4, b4, wo, bo, bm=512):
    grid = (B // bm,)
    full = lambda shape: pl.BlockSpec(shape, lambda i, _n=None: tuple(0 for _ in shape))
    return pl.pallas_call(
        _mlp_kernel,
        grid=grid,
        in_specs=[
            pl.BlockSpec((S, bm, D), lambda i: (0, i, 0)),
            full(w1r.shape), full(b1.shape),
            full(w2.shape), full(b2.shape),
            full(w3.shape), full(b3.shape),
            full(w4.shape), full(b4.shape),
            full(wo.shape), full(bo.shape),
        ],
        out_specs=pl.BlockSpec((bm, 1), lambda i: (i, 0)),
        out_shape=jax.ShapeDtypeStruct((B, 1), jnp.float32),
    )(pooled, w1r, b1, w2, b2, w3, b3, w4, b4, wo, bo)


def kernel(indices, tables, W1, b1, W2, b2, W3, b3, W4, b4, Wo, bo):
    idx4 = indices.astype(jnp.int32).reshape(S, NW, NCHUNK, CHUNK)
    tab_flat = tables.reshape(S * V, D)
    pooled = _sc_pool(tab_flat, idx4)
    w1r = (W1 * (1.0 / L)).reshape(S, D, 256)
    return _mlp(pooled, w1r, b1.reshape(1, 256), W2, b2.reshape(1, 256),
                W3, b3.reshape(1, 128), W4, b4.reshape(1, 128),
                Wo, bo.reshape(1, 1))


# speedup vs baseline: 7.0693x; 7.0693x over previous
import functools

import jax
import jax.numpy as jnp
from jax import lax
from jax.experimental import pallas as pl
from jax.experimental.pallas import tpu as pltpu
from jax.experimental.pallas import tpu_sc as plsc

S, V, D, B, L = 26, 100000, 32, 4096, 20

NC, NS = 2, 16
NW = NC * NS
BPW = B // NW
RPW = BPW * L
CHUNK = 128
NCHUNK = RPW // CHUNK


def _sc_pool_kernel(tab_ref, idx_ref, out_ref, idx_v, rows_v, out_v, sem):
    wid = lax.axis_index("s") * NC + lax.axis_index("c")
    base = wid * BPW

    def slot_body(s, _):
        pltpu.sync_copy(idx_ref.at[s, wid], idx_v)
        s_off = s * V

        def fire(c, _):
            for i in range(CHUNK // 16):
                sl = pl.ds(i * 16, 16)
                idx_v[c, sl] = idx_v[c, sl] + s_off
            pltpu.make_async_copy(
                tab_ref.at[idx_v.at[c]],
                rows_v.at[pl.ds(c * CHUNK, CHUNK)],
                sem,
            ).start()
            return 0

        lax.fori_loop(0, NCHUNK, fire, 0)

        def drain(c, _):
            pltpu.make_async_copy(
                tab_ref.at[idx_v.at[c]],
                rows_v.at[pl.ds(c * CHUNK, CHUNK)],
                sem,
            ).wait()
            return 0

        lax.fori_loop(0, NCHUNK, drain, 0)

        def pool(b, _):
            r0 = b * L
            lo = rows_v[r0, pl.ds(0, 16)]
            hi = rows_v[r0, pl.ds(16, 16)]
            for r in range(1, L):
                lo = lo + rows_v[r0 + r, pl.ds(0, 16)]
                hi = hi + rows_v[r0 + r, pl.ds(16, 16)]
            out_v[b, pl.ds(0, 16)] = lo
            out_v[b, pl.ds(16, 16)] = hi
            return 0

        lax.fori_loop(0, BPW, pool, 0)
        pltpu.sync_copy(out_v, out_ref.at[s, pl.ds(base, BPW)])
        return 0

    lax.fori_loop(0, S, slot_body, 0)


@jax.jit
def _sc_pool(tab_flat, idx4):
    mesh = plsc.VectorSubcoreMesh(core_axis_name="c", subcore_axis_name="s")
    return pl.kernel(
        _sc_pool_kernel,
        out_type=jax.ShapeDtypeStruct((S, B, D), jnp.float32),
        mesh=mesh,
        scratch_types=[
            pltpu.VMEM((NCHUNK, CHUNK), jnp.int32),
            pltpu.VMEM((RPW, D), jnp.float32),
            pltpu.VMEM((BPW, D), jnp.float32),
            pltpu.SemaphoreType.DMA,
        ],
        compiler_params=pltpu.CompilerParams(use_tc_tiling_on_sc=False),
    )(tab_flat, idx4)


def _mlp_kernel(x_ref, w1_ref, b1_ref, w2_ref, b2_ref, w3_ref, b3_ref,
                w4_ref, b4_ref, wo_ref, bo_ref, o_ref):
    x = x_ref[...]
    w1 = w1_ref[...]
    h = jnp.dot(x[0], w1[0], preferred_element_type=jnp.float32)
    for s in range(1, S):
        h = h + jnp.dot(x[s], w1[s], preferred_element_type=jnp.float32)
    h = jnp.maximum(h + b1_ref[...], 0.0)
    h = jnp.maximum(jnp.dot(h, w2_ref[...], preferred_element_type=jnp.float32)
                    + b2_ref[...], 0.0)
    h = jnp.maximum(jnp.dot(h, w3_ref[...], preferred_element_type=jnp.float32)
                    + b3_ref[...], 0.0)
    h = jnp.maximum(jnp.dot(h, w4_ref[...], preferred_element_type=jnp.float32)
                    + b4_ref[...], 0.0)
    y = jnp.dot(h, wo_ref[...], preferred_element_type=jnp.float32) + bo_ref[...]
    o_ref[...] = jax.nn.sigmoid(y)


def _mlp(pooled, w1r, b1, w2, b2, w3, b3, w4, b4, wo, bo, bm=512):
    grid = (B // bm,)
    full = lambda shape: pl.BlockSpec(shape, lambda i, _n=None: tuple(0 for _ in shape))
    return pl.pallas_call(
        _mlp_kernel,
        grid=grid,
        in_specs=[
            pl.BlockSpec((S, bm, D), lambda i: (0, i, 0)),
            full(w1r.shape), full(b1.shape),
            full(w2.shape), full(b2.shape),
            full(w3.shape), full(b3.shape),
            full(w4.shape), full(b4.shape),
            full(wo.shape), full(bo.shape),
        ],
        out_specs=pl.BlockSpec((bm, 1), lambda i: (i, 0)),
        out_shape=jax.ShapeDtypeStruct((B, 1), jnp.float32),
    )(pooled, w1r, b1, w2, b2, w3, b3, w4, b4, wo, bo)


def kernel(indices, tables, W1, b1, W2, b2, W3, b3, W4, b4, Wo, bo):
    idx4 = indices.astype(jnp.int32).reshape(S, NW, NCHUNK, CHUNK)
    tab_flat = tables.reshape(S * V, D)
    pooled = _sc_pool(tab_flat, idx4)
    w1r = (W1 * (1.0 / L)).reshape(S, D, 256)
    return _mlp(pooled, w1r, b1.reshape(1, 256), W2, b2.reshape(1, 256),
                W3, b3.reshape(1, 128), W4, b4.reshape(1, 128),
                Wo, bo.reshape(1, 1))


# speedup vs baseline: 7.5475x; 1.0676x over previous
import jax
import jax.numpy as jnp
from jax import lax
from jax.experimental import pallas as pl
from jax.experimental.pallas import tpu as pltpu
from jax.experimental.pallas import tpu_sc as plsc

S, V, D, B, L = 26, 100000, 32, 4096, 20

NC, NS = 2, 16
NW = NC * NS
BPW = B // NW
CHUNK = 128
NCHUNK = BPW * L // CHUNK
HCHUNK = NCHUNK // 2
HB = BPW // 2
HROWS = HB * L
NPHASE = 2 * S


def _tree_sum(terms):
    while len(terms) > 1:
        nxt = [terms[i] + terms[i + 1] for i in range(0, len(terms) - 1, 2)]
        if len(terms) % 2:
            nxt.append(terms[-1])
        terms = nxt
    return terms[0]


def _sc_pool_kernel(tab_ref, idx_ref, out_ref, idx_v, rows_v, out_v,
                    semg0, semg1, semi):
    wid = lax.axis_index("s") * NC + lax.axis_index("c")
    base = wid * BPW

    def fire_half(sq, hq_static, rbuf, sem):
        ibuf = sq % 2
        s_off = sq * V
        for c in range(HCHUNK):
            row = hq_static * HCHUNK + c
            for i in range(CHUNK // 16):
                sl = pl.ds(i * 16, 16)
                idx_v[ibuf, row, sl] = idx_v[ibuf, row, sl] + s_off
            pltpu.make_async_copy(
                tab_ref.at[idx_v.at[ibuf, row]],
                rows_v.at[rbuf, pl.ds(c * CHUNK, CHUNK)],
                sem,
            ).start()

    def drain_half(rbuf, sem):
        for c in range(HCHUNK):
            pltpu.make_async_copy(
                tab_ref.at[idx_v.at[0, 0]],
                rows_v.at[rbuf, pl.ds(c * CHUNK, CHUNK)],
                sem,
            ).wait()

    def pool_half(rbuf, hq_static):
        obase = hq_static * HB

        def pool2(i, _):
            for k in range(2):
                b = i * 2 + k
                r0 = b * L
                lo = _tree_sum([rows_v[rbuf, r0 + r, pl.ds(0, 16)]
                                for r in range(L)])
                hi = _tree_sum([rows_v[rbuf, r0 + r, pl.ds(16, 16)]
                                for r in range(L)])
                out_v[obase + b, pl.ds(0, 16)] = lo
                out_v[obase + b, pl.ds(16, 16)] = hi
            return 0

        lax.fori_loop(0, HB // 2, pool2, 0)

    pltpu.sync_copy(idx_ref.at[0, wid], idx_v.at[0])
    fire_half(jnp.int32(0), 0, 0, semg0)

    def phase_body(p, _):
        s = p // 2
        h = p % 2

        @pl.when(h == 0)
        def _():
            @pl.when(s + 1 < S)
            def _():
                pltpu.make_async_copy(
                    idx_ref.at[s + 1, wid], idx_v.at[(s + 1) % 2], semi,
                ).start()

            fire_half(s, 1, 1, semg1)
            drain_half(0, semg0)
            pool_half(0, 0)

        @pl.when(h == 1)
        def _():
            @pl.when(s + 1 < S)
            def _():
                pltpu.make_async_copy(
                    idx_ref.at[s + 1, wid], idx_v.at[(s + 1) % 2], semi,
                ).wait()
                fire_half(s + 1, 0, 0, semg0)

            drain_half(1, semg1)
            pool_half(1, 1)
            pltpu.sync_copy(out_v, out_ref.at[s, pl.ds(base, BPW)])

        return 0

    lax.fori_loop(0, NPHASE, phase_body, 0)


@jax.jit
def _sc_pool(tab_flat, idx4):
    mesh = plsc.VectorSubcoreMesh(core_axis_name="c", subcore_axis_name="s")
    return pl.kernel(
        _sc_pool_kernel,
        out_type=jax.ShapeDtypeStruct((S, B, D), jnp.float32),
        mesh=mesh,
        scratch_types=[
            pltpu.VMEM((2, NCHUNK, CHUNK), jnp.int32),
            pltpu.VMEM((2, HROWS, D), jnp.float32),
            pltpu.VMEM((BPW, D), jnp.float32),
            pltpu.SemaphoreType.DMA,
            pltpu.SemaphoreType.DMA,
            pltpu.SemaphoreType.DMA,
        ],
        compiler_params=pltpu.CompilerParams(use_tc_tiling_on_sc=False),
    )(tab_flat, idx4)


def _mlp_kernel(x_ref, w1_ref, b1_ref, w2_ref, b2_ref, w3_ref, b3_ref,
                w4_ref, b4_ref, wo_ref, bo_ref, o_ref):
    x = x_ref[...]
    w1 = w1_ref[...]
    h = jnp.dot(x[0], w1[0], preferred_element_type=jnp.float32)
    for s in range(1, S):
        h = h + jnp.dot(x[s], w1[s], preferred_element_type=jnp.float32)
    h = jnp.maximum(h + b1_ref[...], 0.0)
    h = jnp.maximum(jnp.dot(h, w2_ref[...], preferred_element_type=jnp.float32)
                    + b2_ref[...], 0.0)
    h = jnp.maximum(jnp.dot(h, w3_ref[...], preferred_element_type=jnp.float32)
                    + b3_ref[...], 0.0)
    h = jnp.maximum(jnp.dot(h, w4_ref[...], preferred_element_type=jnp.float32)
                    + b4_ref[...], 0.0)
    y = jnp.dot(h, wo_ref[...], preferred_element_type=jnp.float32) + bo_ref[...]
    o_ref[...] = jax.nn.sigmoid(y)


def _mlp(pooled, w1r, b1, w2, b2, w3, b3, w4, b4, wo, bo, bm=512):
    grid = (B // bm,)
    full = lambda shape: pl.BlockSpec(shape, lambda i: tuple(0 for _ in shape))
    return pl.pallas_call(
        _mlp_kernel,
        grid=grid,
        in_specs=[
            pl.BlockSpec((S, bm, D), lambda i: (0, i, 0)),
            full(w1r.shape), full(b1.shape),
            full(w2.shape), full(b2.shape),
            full(w3.shape), full(b3.shape),
            full(w4.shape), full(b4.shape),
            full(wo.shape), full(bo.shape),
        ],
        out_specs=pl.BlockSpec((bm, 1), lambda i: (i, 0)),
        out_shape=jax.ShapeDtypeStruct((B, 1), jnp.float32),
    )(pooled, w1r, b1, w2, b2, w3, b3, w4, b4, wo, bo)


def kernel(indices, tables, W1, b1, W2, b2, W3, b3, W4, b4, Wo, bo):
    idx4 = indices.astype(jnp.int32).reshape(S, NW, NCHUNK, CHUNK)
    tab_flat = tables.reshape(S * V, D)
    pooled = _sc_pool(tab_flat, idx4)
    w1r = (W1 * (1.0 / L)).reshape(S, D, 256)
    return _mlp(pooled, w1r, b1.reshape(1, 256), W2, b2.reshape(1, 256),
                W3, b3.reshape(1, 128), W4, b4.reshape(1, 128),
                Wo, bo.reshape(1, 1))
